# baseline (speedup 1.0000x reference)
import functools

import jax
import jax.numpy as jnp
from jax.experimental import pallas as pl
from jax.experimental.pallas import tpu as pltpu

T = 2048
D = 2048
E = 8
F = 1024
FC = 512
NF = F // FC


def _router_body(x_ref, rw_ref, cw_ref):
    logits = jax.lax.dot_general(
        x_ref[...], rw_ref[...], (((1,), (1,)), ((), ())),
        precision=jax.lax.Precision.DEFAULT,
        preferred_element_type=jnp.float32)
    m = jnp.max(logits, axis=-1, keepdims=True)
    ex = jnp.exp(logits - m)
    probs = ex / jnp.sum(ex, axis=-1, keepdims=True)
    iota = jax.lax.broadcasted_iota(jnp.int32, (T, E), 1)
    m1 = jnp.max(probs, axis=-1, keepdims=True)
    i1 = jnp.min(jnp.where(probs == m1, iota, E), axis=-1, keepdims=True)
    p2 = jnp.where(iota == i1, -jnp.inf, probs)
    m2 = jnp.max(p2, axis=-1, keepdims=True)
    i2 = jnp.min(jnp.where(p2 == m2, iota, E), axis=-1, keepdims=True)
    s = m1 + m2
    cw_ref[...] = (jnp.where(iota == i1, m1 / s, 0.0)
                   + jnp.where(iota == i2, m2 / s, 0.0))


def _moe_body(xb_ref, cw_ref, wg_ref, wv_ref, w2_ref, out_ref):
    e = pl.program_id(0)
    j = pl.program_id(1)

    @pl.when((e == 0) & (j == 0))
    def _():
        out_ref[...] = jnp.zeros_like(out_ref)

    xb = xb_ref[...]
    gate = jax.lax.dot_general(
        xb, wg_ref[0], (((1,), (1,)), ((), ())),
        preferred_element_type=jnp.float32)
    up = jax.lax.dot_general(
        xb, wv_ref[0], (((1,), (1,)), ((), ())),
        preferred_element_type=jnp.float32)
    act = gate * jax.nn.sigmoid(gate) * up

    lane = jax.lax.broadcasted_iota(jnp.int32, (T, E), 1)
    cw_col = jnp.sum(jnp.where(lane == e, cw_ref[...], 0.0),
                     axis=1, keepdims=True)
    act_w = (act * cw_col).astype(jnp.bfloat16)

    contrib = jax.lax.dot_general(
        act_w, w2_ref[0], (((1,), (1,)), ((), ())),
        preferred_element_type=jnp.float32)
    out_ref[...] += contrib


@jax.jit
def kernel(hidden_states, router_weight, ws, w2s):
    x = hidden_states.reshape(T, D)
    cw = pl.pallas_call(
        _router_body,
        out_shape=jax.ShapeDtypeStruct((T, E), jnp.float32),
        in_specs=[pl.BlockSpec((T, D), lambda: (0, 0)),
                  pl.BlockSpec((E, D), lambda: (0, 0))],
        out_specs=pl.BlockSpec((T, E), lambda: (0, 0)),
    )(x, router_weight)

    xb = x.astype(jnp.bfloat16)
    wsb = ws.astype(jnp.bfloat16)
    w2b = w2s.astype(jnp.bfloat16)

    out = pl.pallas_call(
        _moe_body,
        grid=(E, NF),
        out_shape=jax.ShapeDtypeStruct((T, D), jnp.float32),
        in_specs=[
            pl.BlockSpec((T, D), lambda e, j: (0, 0)),
            pl.BlockSpec((T, E), lambda e, j: (0, 0)),
            pl.BlockSpec((1, FC, D), lambda e, j: (e, j, 0)),
            pl.BlockSpec((1, FC, D), lambda e, j: (e, j + NF, 0)),
            pl.BlockSpec((1, D, FC), lambda e, j: (e, 0, j)),
        ],
        out_specs=pl.BlockSpec((T, D), lambda e, j: (0, 0)),
        compiler_params=pltpu.CompilerParams(
            dimension_semantics=("arbitrary", "arbitrary"),
        ),
    )(xb, cw, wsb, wsb, w2b)
    return out.reshape(hidden_states.shape)


# speedup vs baseline: 1.3698x; 1.3698x over previous
import functools

import jax
import jax.numpy as jnp
from jax.experimental import pallas as pl
from jax.experimental.pallas import tpu as pltpu

T = 2048
D = 2048
E = 8
F = 1024
FC = 256
NF = F // FC


def _router_body(x_ref, rw_ref, cw_ref):
    logits = jax.lax.dot_general(
        x_ref[...], rw_ref[...], (((1,), (1,)), ((), ())),
        precision=jax.lax.Precision.DEFAULT,
        preferred_element_type=jnp.float32)
    m = jnp.max(logits, axis=-1, keepdims=True)
    ex = jnp.exp(logits - m)
    probs = ex / jnp.sum(ex, axis=-1, keepdims=True)
    iota = jax.lax.broadcasted_iota(jnp.int32, (T, E), 1)
    m1 = jnp.max(probs, axis=-1, keepdims=True)
    i1 = jnp.min(jnp.where(probs == m1, iota, E), axis=-1, keepdims=True)
    p2 = jnp.where(iota == i1, -jnp.inf, probs)
    m2 = jnp.max(p2, axis=-1, keepdims=True)
    i2 = jnp.min(jnp.where(p2 == m2, iota, E), axis=-1, keepdims=True)
    s = m1 + m2
    cw_ref[...] = (jnp.where(iota == i1, m1 / s, 0.0)
                   + jnp.where(iota == i2, m2 / s, 0.0))


def _moe_body(xb_ref, cw_ref, wg_ref, wv_ref, w2_ref, out_ref):
    e = pl.program_id(0)
    j = pl.program_id(1)

    @pl.when((e == 0) & (j == 0))
    def _():
        out_ref[...] = jnp.zeros_like(out_ref)

    xb = xb_ref[...]
    gate = jax.lax.dot_general(
        xb, wg_ref[0], (((1,), (1,)), ((), ())),
        precision=jax.lax.Precision.DEFAULT,
        preferred_element_type=jnp.float32)
    up = jax.lax.dot_general(
        xb, wv_ref[0], (((1,), (1,)), ((), ())),
        precision=jax.lax.Precision.DEFAULT,
        preferred_element_type=jnp.float32)
    act = gate * jax.nn.sigmoid(gate) * up

    lane = jax.lax.broadcasted_iota(jnp.int32, (T, E), 1)
    cw_col = jnp.sum(jnp.where(lane == e, cw_ref[...], 0.0),
                     axis=1, keepdims=True)
    act_w = act * cw_col

    contrib = jax.lax.dot_general(
        act_w, w2_ref[0], (((1,), (1,)), ((), ())),
        precision=jax.lax.Precision.DEFAULT,
        preferred_element_type=jnp.float32)
    out_ref[...] += contrib


@jax.jit
def kernel(hidden_states, router_weight, ws, w2s):
    x = hidden_states.reshape(T, D)
    cw = pl.pallas_call(
        _router_body,
        out_shape=jax.ShapeDtypeStruct((T, E), jnp.float32),
        in_specs=[pl.BlockSpec((T, D), lambda: (0, 0)),
                  pl.BlockSpec((E, D), lambda: (0, 0))],
        out_specs=pl.BlockSpec((T, E), lambda: (0, 0)),
    )(x, router_weight)

    out = pl.pallas_call(
        _moe_body,
        grid=(E, NF),
        out_shape=jax.ShapeDtypeStruct((T, D), jnp.float32),
        in_specs=[
            pl.BlockSpec((T, D), lambda e, j: (0, 0)),
            pl.BlockSpec((T, E), lambda e, j: (0, 0)),
            pl.BlockSpec((1, FC, D), lambda e, j: (e, j, 0)),
            pl.BlockSpec((1, FC, D), lambda e, j: (e, j + NF, 0)),
            pl.BlockSpec((1, D, FC), lambda e, j: (e, 0, j)),
        ],
        out_specs=pl.BlockSpec((T, D), lambda e, j: (0, 0)),
        compiler_params=pltpu.CompilerParams(
            dimension_semantics=("arbitrary", "arbitrary"),
        ),
    )(x, cw, ws, ws, w2s)
    return out.reshape(hidden_states.shape)
